# baseline (speedup 1.0000x reference)
import functools

import jax
import jax.numpy as jnp
from jax import lax
from jax.experimental import pallas as pl
from jax.experimental.pallas import tpu as pltpu
from jax.experimental.pallas import tpu_sc as plsc

_CP = 232
_COLS = 10


def _sc_gather_rows(embedding_table, ids, B, T, H):
    V, D = embedding_table.shape
    mesh = plsc.VectorSubcoreMesh(core_axis_name="c", subcore_axis_name="s")

    def body(table_hbm, ids_hbm, rows_hbm, idx_v, gath_v, sem):
        wid = lax.axis_index("s") * plsc.get_sparse_core_info().num_cores + \
            lax.axis_index("c")

        @pl.when(wid == 0)
        def _():
            pltpu.sync_copy(ids_hbm, idx_v)
            pltpu.async_copy(table_hbm.at[idx_v], gath_v, sem).wait()
            for seg in range(B * T):
                b, t = divmod(seg, T)
                pltpu.sync_copy(
                    gath_v.at[pl.ds(b, 1), pl.ds(t * H, H)],
                    rows_hbm.at[pl.ds(seg, 1)],
                )

    k = functools.partial(
        pl.kernel,
        out_type=jax.ShapeDtypeStruct((B * T, H), jnp.float32),
        mesh=mesh,
        scratch_types=[
            pltpu.VMEM((B,), jnp.int32),
            pltpu.VMEM((B, D), jnp.float32),
            pltpu.SemaphoreType.DMA,
        ],
    )(body)
    return k(embedding_table, ids)


def _add_body(ids_ref, hid_ref, rows_ref, gate_ref, out_ref, inb, outb, isem, osem):
    B, T, P, H = hid_ref.shape
    NSEG = B * T
    chunks = list(range(_COLS))

    g = jnp.tanh(gate_ref[...])

    def transfers(seg, inward):
        b, t = divmod(seg, T)
        pg = seg % 2
        for c in chunks:
            sl = pl.ds(c * 128, 128)
            if inward:
                yield pltpu.make_async_copy(
                    hid_ref.at[b, t, slice(None), sl],
                    inb.at[pg, slice(None), sl],
                    isem.at[pg],
                )
            else:
                yield pltpu.make_async_copy(
                    outb.at[pg, slice(None), sl],
                    out_ref.at[b, t, slice(None), sl],
                    osem.at[pg],
                )

    def start(seg, inward):
        for c in transfers(seg, inward):
            c.start()

    def wait(seg, inward):
        for c in transfers(seg, inward):
            c.wait()

    start(0, True)
    start(1, True)
    for seg in range(NSEG):
        pg = seg % 2
        wait(seg, True)
        if seg >= 2:
            wait(seg - 2, False)
        outb[pg] = inb[pg] + rows_ref[pl.ds(seg, 1)] * g
        start(seg, False)
        if seg + 2 < NSEG:
            start(seg + 2, True)
    wait(NSEG - 2, False)
    wait(NSEG - 1, False)


def kernel(hidden_state, aspect_ratio_ids, embedding_table, gate):
    B, T, P, H = hidden_state.shape
    ids = aspect_ratio_ids.astype(jnp.int32)
    gate2d = gate.reshape(1, 1)

    rows = _sc_gather_rows(embedding_table, ids, B, T, H)

    grid_spec = pltpu.PrefetchScalarGridSpec(
        num_scalar_prefetch=1,
        grid=(1,),
        in_specs=[
            pl.BlockSpec(memory_space=pl.ANY),
            pl.BlockSpec((B * T, H), lambda i, ids_ref: (0, 0)),
            pl.BlockSpec((1, 1), lambda i, ids_ref: (0, 0)),
        ],
        out_specs=pl.BlockSpec(memory_space=pl.ANY),
        scratch_shapes=[
            pltpu.VMEM((2, P, H), jnp.float32),
            pltpu.VMEM((2, P, H), jnp.float32),
            pltpu.SemaphoreType.DMA((2,)),
            pltpu.SemaphoreType.DMA((2,)),
        ],
    )
    return pl.pallas_call(
        _add_body,
        grid_spec=grid_spec,
        out_shape=jax.ShapeDtypeStruct((B, T, P, H), hidden_state.dtype),
    )(ids, hidden_state, rows, gate2d)
